# Initial kernel scaffold; baseline (speedup 1.0000x reference)
import jax
import jax.numpy as jnp
from jax.experimental import pallas as pl


def kernel(x, edge_index, W1, b1, W2, b2, W3, b3):
    raise NotImplementedError("write your pallas kernel here")


# speedup vs baseline: 10.3189x; 10.3189x over previous
import functools

import jax
import jax.numpy as jnp
from jax import lax
from jax.experimental import pallas as pl
from jax.experimental.pallas import tpu as pltpu
from jax.experimental.pallas import tpu_sc as plsc

N = 10000
E = 320000
NC, NS = 2, 16
NW = NC * NS
EPW = E // NW
C = 125
NCHUNK = EPW // C
RB = 624
REM = N - RB * NS
RZ = RB + REM
F1, F2, F3 = 128, 64, 64
CW = 16

_SC_PARAMS = pltpu.CompilerParams(use_tc_tiling_on_sc=False)


def _mesh():
    return plsc.VectorSubcoreMesh(core_axis_name="c", subcore_axis_name="s")


def _zero_acc(sid, z_h, acc):
    pltpu.sync_copy(z_h.at[pl.ds(0, RB)], acc.at[pl.ds(sid * RB, RB)])

    @pl.when(sid == 0)
    def _():
        pltpu.sync_copy(z_h.at[pl.ds(0, REM)], acc.at[pl.ds(RB * NS, REM)])


def _copy_out(sid, acc, out2d):
    rows = pl.ds(sid * RB, RB)
    pltpu.sync_copy(acc.at[rows], out2d.at[rows])

    @pl.when(sid == 0)
    def _():
        tail = pl.ds(RB * NS, REM)
        pltpu.sync_copy(acc.at[tail], out2d.at[tail])


def _sc_degrees(src3, dst3, ones_h, zeros_h):
    @functools.partial(
        pl.kernel,
        out_type=(jax.ShapeDtypeStruct((NC, N, CW), jnp.float32),
                  jax.ShapeDtypeStruct((NC, N, CW), jnp.float32)),
        mesh=_mesh(),
        compiler_params=_SC_PARAMS,
        scratch_types=[
            pltpu.VMEM((NCHUNK, C), jnp.int32),
            pltpu.VMEM((NCHUNK, C), jnp.int32),
            pltpu.VMEM((C, CW), jnp.float32),
            pltpu.VMEM_SHARED((N, CW), jnp.float32),
            pltpu.VMEM_SHARED((N, CW), jnp.float32),
        ],
    )
    def k(src_h, dst_h, ones_hr, z_h, dego_h, degi_h,
          src_v, dst_v, ones_v, dego_s, degi_s):
        cid = lax.axis_index("c")
        sid = lax.axis_index("s")
        wid = sid * NC + cid
        _zero_acc(sid, z_h, dego_s)
        _zero_acc(sid, z_h, degi_s)
        pltpu.sync_copy(ones_hr, ones_v)
        pltpu.sync_copy(src_h.at[wid], src_v)
        pltpu.sync_copy(dst_h.at[wid], dst_v)
        plsc.subcore_barrier()

        @pl.loop(0, NCHUNK)
        def _(j):
            pltpu.sync_copy(ones_v, dego_s.at[src_v.at[j]], add=True)
            pltpu.sync_copy(ones_v, degi_s.at[dst_v.at[j]], add=True)

        plsc.subcore_barrier()
        _copy_out(sid, dego_s, dego_h.at[cid])
        _copy_out(sid, degi_s, degi_h.at[cid])

    return k(src3, dst3, ones_h, zeros_h)


def _sc_propagate(table, gidx3, sidx3, zeros_h, D):
    @functools.partial(
        pl.kernel,
        out_type=jax.ShapeDtypeStruct((NC, N, D), jnp.float32),
        mesh=_mesh(),
        compiler_params=_SC_PARAMS,
        scratch_types=[
            pltpu.VMEM((NCHUNK, C), jnp.int32),
            pltpu.VMEM((NCHUNK, C), jnp.int32),
            pltpu.VMEM((C, D), jnp.float32),
            pltpu.VMEM_SHARED((N, D), jnp.float32),
            pltpu.SemaphoreType.DMA,
        ],
    )
    def k(t_h, g_h, s_h, z_h, agg_h,
          g_v, s_v, rows_v, acc_s, sem1):
        cid = lax.axis_index("c")
        sid = lax.axis_index("s")
        wid = sid * NC + cid
        _zero_acc(sid, z_h, acc_s)
        pltpu.sync_copy(g_h.at[wid], g_v)
        pltpu.sync_copy(s_h.at[wid], s_v)
        plsc.subcore_barrier()

        @pl.loop(0, NCHUNK)
        def _(j):
            pltpu.async_copy(t_h.at[g_v.at[j]], rows_v, sem1).wait()
            pltpu.sync_copy(rows_v, acc_s.at[s_v.at[j]], add=True)

        plsc.subcore_barrier()
        _copy_out(sid, acc_s, agg_h.at[cid])

    return k(table, gidx3, sidx3, zeros_h)


def _sc_layer1(table1, src3, dst3, z128):
    return _sc_propagate(table1, src3, dst3, z128, F1)


def _sc_layer2(table2, src3, dst3, z64):
    return _sc_propagate(table2, src3, dst3, z64, F2)


def _sc_cpass(tnd, src3, dst3, z16):
    return _sc_propagate(tnd, dst3, src3, z16, CW)


def _norms(do_ref, di_ref):
    dego = do_ref[0, :, 0:1] + do_ref[1, :, 0:1]
    degi = di_ref[0, :, 0:1] + di_ref[1, :, 0:1]
    ns = jnp.where(dego > 0, lax.rsqrt(dego), 0.0)
    nd = jnp.where(degi > 0, lax.rsqrt(degi), 0.0)
    return ns, nd


def _dot(a, b):
    return lax.dot_general(a, b, (((1,), (0,)), ((), ())),
                           precision=lax.Precision.HIGHEST,
                           preferred_element_type=jnp.float32)


def _tc_prep(x, W1, dego_p, degi_p):
    def body(x_ref, w_ref, do_ref, di_ref, t1_ref, tnd_ref):
        ns, nd = _norms(do_ref, di_ref)
        t1_ref[...] = _dot(x_ref[...] * ns, w_ref[...])
        tnd_ref[...] = jnp.broadcast_to(nd, (N, CW))

    return pl.pallas_call(
        body,
        out_shape=(jax.ShapeDtypeStruct((N, F1), jnp.float32),
                   jax.ShapeDtypeStruct((N, CW), jnp.float32)),
    )(x, W1, dego_p, degi_p)


def _tc_mid(agg1_p, dego_p, degi_p, b1r, W2):
    def body(a_ref, do_ref, di_ref, b_ref, w_ref, t2_ref):
        ns, nd = _norms(do_ref, di_ref)
        h1 = jnp.maximum((a_ref[0] + a_ref[1]) * nd + b_ref[...], 0.0)
        t2_ref[...] = _dot(h1 * ns, w_ref[...])

    return pl.pallas_call(
        body,
        out_shape=jax.ShapeDtypeStruct((N, F2), jnp.float32),
    )(agg1_p, dego_p, degi_p, b1r, W2)


def _tc_final(agg2_p, c_p, dego_p, degi_p, b2r, W3, b3r):
    def body(a_ref, c_ref, do_ref, di_ref, b2_ref, w_ref, b3_ref, o_ref):
        ns, nd = _norms(do_ref, di_ref)
        h2 = jnp.maximum((a_ref[0] + a_ref[1]) * nd + b2_ref[...], 0.0)
        c = c_ref[0, :, 0:1] + c_ref[1, :, 0:1]
        wv = c * ns * (1.0 / N)
        pooled = jnp.sum(h2 * wv, axis=0, keepdims=True)
        o_ref[...] = _dot(pooled, w_ref[...]) + b3_ref[...]

    return pl.pallas_call(
        body,
        out_shape=jax.ShapeDtypeStruct((1, F3), jnp.float32),
    )(agg2_p, c_p, dego_p, degi_p, b2r, W3, b3r)


def kernel(x, edge_index, W1, b1, W2, b2, W3, b3):
    src3 = edge_index[0].reshape(NW, NCHUNK, C)
    dst3 = edge_index[1].reshape(NW, NCHUNK, C)
    ones16 = jnp.ones((C, CW), jnp.float32)
    z16 = jnp.zeros((RZ, CW), jnp.float32)
    z64 = jnp.zeros((RZ, F2), jnp.float32)
    z128 = jnp.zeros((RZ, F1), jnp.float32)

    dego_p, degi_p = _sc_degrees(src3, dst3, ones16, z16)
    t1, tnd = _tc_prep(x, W1, dego_p, degi_p)
    agg1_p = _sc_layer1(t1, src3, dst3, z128)
    c_p = _sc_cpass(tnd, src3, dst3, z16)
    t2 = _tc_mid(agg1_p, dego_p, degi_p, b1.reshape(1, F1), W2)
    agg2_p = _sc_layer2(t2, src3, dst3, z64)
    return _tc_final(agg2_p, c_p, dego_p, degi_p,
                     b2.reshape(1, F2), W3, b3.reshape(1, F3))


# speedup vs baseline: 13.0216x; 1.2619x over previous
import functools

import jax
import jax.numpy as jnp
from jax import lax
from jax.experimental import pallas as pl
from jax.experimental.pallas import tpu as pltpu
from jax.experimental.pallas import tpu_sc as plsc

N = 10000
E = 320000
NC, NS = 2, 16
NW = NC * NS
EPW = E // NW
C = 125
NCHUNK = EPW // C
RB = 624
REM = N - RB * NS
RZ = RB + REM
F1, F2, F3 = 128, 64, 64
CW = 16

_SC_PARAMS = pltpu.CompilerParams(use_tc_tiling_on_sc=False)


def _mesh():
    return plsc.VectorSubcoreMesh(core_axis_name="c", subcore_axis_name="s")


def _zero_acc(sid, z_h, acc):
    pltpu.sync_copy(z_h.at[pl.ds(0, RB)], acc.at[pl.ds(sid * RB, RB)])

    @pl.when(sid == 0)
    def _():
        pltpu.sync_copy(z_h.at[pl.ds(0, REM)], acc.at[pl.ds(RB * NS, REM)])


def _copy_out(sid, acc, out2d):
    rows = pl.ds(sid * RB, RB)
    pltpu.sync_copy(acc.at[rows], out2d.at[rows])

    @pl.when(sid == 0)
    def _():
        tail = pl.ds(RB * NS, REM)
        pltpu.sync_copy(acc.at[tail], out2d.at[tail])


def _sc_degrees(src3, dst3, ones_h, zeros_h):
    @functools.partial(
        pl.kernel,
        out_type=(jax.ShapeDtypeStruct((NC, N, CW), jnp.float32),
                  jax.ShapeDtypeStruct((NC, N, CW), jnp.float32)),
        mesh=_mesh(),
        compiler_params=_SC_PARAMS,
        scratch_types=[
            pltpu.VMEM((NCHUNK, C), jnp.int32),
            pltpu.VMEM((NCHUNK, C), jnp.int32),
            pltpu.VMEM((C, CW), jnp.float32),
            pltpu.VMEM_SHARED((N, CW), jnp.float32),
            pltpu.VMEM_SHARED((N, CW), jnp.float32),
            pltpu.SemaphoreType.DMA,
        ],
    )
    def k(src_h, dst_h, ones_hr, z_h, dego_h, degi_h,
          src_v, dst_v, ones_v, dego_s, degi_s, ssem):
        cid = lax.axis_index("c")
        sid = lax.axis_index("s")
        wid = sid * NC + cid
        _zero_acc(sid, z_h, dego_s)
        _zero_acc(sid, z_h, degi_s)
        pltpu.sync_copy(ones_hr, ones_v)
        pltpu.sync_copy(src_h.at[wid], src_v)
        pltpu.sync_copy(dst_h.at[wid], dst_v)
        plsc.subcore_barrier()

        @pl.loop(0, NCHUNK, step=4)
        def _(j):
            for o in range(4):
                pltpu.async_copy(ones_v, dego_s.at[src_v.at[j + o]], ssem,
                                 add=True)
                pltpu.async_copy(ones_v, degi_s.at[dst_v.at[j + o]], ssem,
                                 add=True)
            for o in range(8):
                pltpu.make_async_copy(ones_v, dego_s.at[src_v.at[0]],
                                      ssem).wait()

        plsc.subcore_barrier()
        _copy_out(sid, dego_s, dego_h.at[cid])
        _copy_out(sid, degi_s, degi_h.at[cid])

    return k(src3, dst3, ones_h, zeros_h)


def _sc_propagate(tables, gidx3, sidx3, zeros_list, Ds, K, swaps):
    NT = len(tables)
    NB = NCHUNK // K
    assert NCHUNK % K == 0 and NB % 2 == 0

    bufs_types = [pltpu.VMEM((C, D), jnp.float32)
                  for _ in range(2) for D in Ds for _k in range(K)]
    acc_types = [pltpu.VMEM_SHARED((N, D), jnp.float32) for D in Ds]
    sem_types = [pltpu.SemaphoreType.DMA] * (4 * NT)

    @functools.partial(
        pl.kernel,
        out_type=tuple(jax.ShapeDtypeStruct((NC, N, D), jnp.float32) for D in Ds),
        mesh=_mesh(),
        compiler_params=_SC_PARAMS,
        scratch_types=[
            pltpu.VMEM((NCHUNK, C), jnp.int32),
            pltpu.VMEM((NCHUNK, C), jnp.int32),
        ] + bufs_types + acc_types + sem_types,
    )
    def k(*refs):
        t_h = refs[:NT]
        g_h, s_h = refs[NT], refs[NT + 1]
        z_h = refs[NT + 2:NT + 2 + NT]
        agg_h = refs[NT + 2 + NT:NT + 2 + 2 * NT]
        g_v, s_v = refs[3 * NT + 2], refs[3 * NT + 3]
        p = 3 * NT + 4
        bufs = [[[refs[p + (g * NT + i) * K + k_] for k_ in range(K)]
                 for i in range(NT)] for g in range(2)]
        p += 2 * NT * K
        accs = refs[p:p + NT]
        p += NT
        gsem = [[refs[p + g * NT + i] for i in range(NT)] for g in range(2)]
        p += 2 * NT
        ssem = [[refs[p + g * NT + i] for i in range(NT)] for g in range(2)]

        cid = lax.axis_index("c")
        sid = lax.axis_index("s")
        wid = sid * NC + cid
        for i in range(NT):
            _zero_acc(sid, z_h[i], accs[i])
        pltpu.sync_copy(g_h.at[wid], g_v)
        pltpu.sync_copy(s_h.at[wid], s_v)
        plsc.subcore_barrier()

        def gv(i):
            return s_v if swaps[i] else g_v

        def sv(i):
            return g_v if swaps[i] else s_v

        def gather(i, j, grp, slot):
            pltpu.async_copy(t_h[i].at[gv(i).at[j]], bufs[grp][i][slot],
                             gsem[grp][i])

        def scatter(i, j, grp, slot):
            pltpu.async_copy(bufs[grp][i][slot], accs[i].at[sv(i).at[j]],
                             ssem[grp][i], add=True)

        def wait_g(grp, i):
            pltpu.make_async_copy(t_h[i].at[gv(i).at[0]], bufs[grp][i][0],
                                  gsem[grp][i]).wait()

        def wait_s(grp, i):
            pltpu.make_async_copy(bufs[grp][i][0], accs[i].at[sv(i).at[0]],
                                  ssem[grp][i]).wait()

        for k_ in range(K):
            for i in range(NT):
                gather(i, k_, 0, k_)
        for k_ in range(K):
            for i in range(NT):
                gather(i, K + k_, 1, k_)

        @pl.loop(0, NB, step=2)
        def _(b):
            base = b * K
            for k_ in range(K):
                for i in range(NT):
                    wait_g(0, i)
            for k_ in range(K):
                for i in range(NT):
                    scatter(i, base + k_, 0, k_)
            for k_ in range(K):
                for i in range(NT):
                    wait_g(1, i)
            for k_ in range(K):
                for i in range(NT):
                    scatter(i, base + K + k_, 1, k_)

            @pl.when(b + 2 < NB)
            def _():
                for k_ in range(K):
                    for i in range(NT):
                        wait_s(0, i)
                for k_ in range(K):
                    for i in range(NT):
                        gather(i, base + 2 * K + k_, 0, k_)
                for k_ in range(K):
                    for i in range(NT):
                        wait_s(1, i)
                for k_ in range(K):
                    for i in range(NT):
                        gather(i, base + 3 * K + k_, 1, k_)

        for k_ in range(K):
            for i in range(NT):
                wait_s(0, i)
                wait_s(1, i)
        plsc.subcore_barrier()
        for i in range(NT):
            _copy_out(sid, accs[i], agg_h[i].at[cid])

    return k(*tables, gidx3, sidx3, *zeros_list)


def _sc_layer1_half(table_half, src3, dst3, z64):
    return _sc_propagate([table_half], src3, dst3, [z64], [F2], 4, [False])[0]


def _sc_layer2c(table2, tnd, src3, dst3, z64, z16):
    return _sc_propagate([table2, tnd], src3, dst3, [z64, z16],
                         [F2, CW], 2, [False, True])


def _norms(do_ref, di_ref):
    dego = do_ref[0, :, 0:1] + do_ref[1, :, 0:1]
    degi = di_ref[0, :, 0:1] + di_ref[1, :, 0:1]
    ns = jnp.where(dego > 0, lax.rsqrt(dego), 0.0)
    nd = jnp.where(degi > 0, lax.rsqrt(degi), 0.0)
    return ns, nd


def _dot(a, b):
    ah = a.astype(jnp.bfloat16)
    al = (a - ah.astype(jnp.float32)).astype(jnp.bfloat16)
    bh = b.astype(jnp.bfloat16)
    bl = (b - bh.astype(jnp.float32)).astype(jnp.bfloat16)

    def d(u, v):
        return lax.dot_general(u, v, (((1,), (0,)), ((), ())),
                               preferred_element_type=jnp.float32)

    return d(ah, bh) + d(ah, bl) + d(al, bh)


G = 10
BR = N // G


def _bs(shape, im):
    return pl.BlockSpec(shape, im)


def _row(i):
    return (i, 0)


def _prow(i):
    return (0, i, 0)


def _full(i):
    return (0, 0)


def _tc_prep(x, W1a, W1b, dego_p, degi_p):
    def body(x_ref, wa_ref, wb_ref, do_ref, di_ref, t1a_ref, t1b_ref, tnd_ref):
        ns, nd = _norms(do_ref, di_ref)
        xs = x_ref[...] * ns
        t1a_ref[...] = _dot(xs, wa_ref[...])
        t1b_ref[...] = _dot(xs, wb_ref[...])
        tnd_ref[...] = jnp.broadcast_to(nd, (BR, CW))

    return pl.pallas_call(
        body,
        grid=(G,),
        in_specs=[_bs((BR, F1), _row), _bs((F1, F2), _full), _bs((F1, F2), _full),
                  _bs((NC, BR, CW), _prow), _bs((NC, BR, CW), _prow)],
        out_specs=(_bs((BR, F2), _row), _bs((BR, F2), _row), _bs((BR, CW), _row)),
        out_shape=(jax.ShapeDtypeStruct((N, F2), jnp.float32),
                   jax.ShapeDtypeStruct((N, F2), jnp.float32),
                   jax.ShapeDtypeStruct((N, CW), jnp.float32)),
    )(x, W1a, W1b, dego_p, degi_p)


def _tc_mid(agg1a_p, agg1b_p, dego_p, degi_p, b1ar, b1br, W2a, W2b):
    def body(aa_ref, ab_ref, do_ref, di_ref, ba_ref, bb_ref, wa_ref, wb_ref,
             t2_ref):
        ns, nd = _norms(do_ref, di_ref)
        h1a = jnp.maximum((aa_ref[0] + aa_ref[1]) * nd + ba_ref[...], 0.0)
        h1b = jnp.maximum((ab_ref[0] + ab_ref[1]) * nd + bb_ref[...], 0.0)
        t2_ref[...] = _dot(h1a * ns, wa_ref[...]) + _dot(h1b * ns, wb_ref[...])

    return pl.pallas_call(
        body,
        grid=(G,),
        in_specs=[_bs((NC, BR, F2), _prow), _bs((NC, BR, F2), _prow),
                  _bs((NC, BR, CW), _prow), _bs((NC, BR, CW), _prow),
                  _bs((1, F2), _full), _bs((1, F2), _full),
                  _bs((F2, F2), _full), _bs((F2, F2), _full)],
        out_specs=_bs((BR, F2), _row),
        out_shape=jax.ShapeDtypeStruct((N, F2), jnp.float32),
    )(agg1a_p, agg1b_p, dego_p, degi_p, b1ar, b1br, W2a, W2b)


def _tc_final(agg2_p, c_p, dego_p, degi_p, b2r, W3, b3r):
    def body(a_ref, c_ref, do_ref, di_ref, b2_ref, w_ref, b3_ref, o_ref,
             acc_ref):
        i = pl.program_id(0)

        @pl.when(i == 0)
        def _():
            acc_ref[...] = jnp.zeros((1, F2), jnp.float32)

        ns, nd = _norms(do_ref, di_ref)
        h2 = jnp.maximum((a_ref[0] + a_ref[1]) * nd + b2_ref[...], 0.0)
        c = c_ref[0, :, 0:1] + c_ref[1, :, 0:1]
        wv = c * ns * (1.0 / N)
        acc_ref[...] += jnp.sum(h2 * wv, axis=0, keepdims=True)

        @pl.when(i == G - 1)
        def _():
            o_ref[...] = _dot(acc_ref[...], w_ref[...]) + b3_ref[...]

    return pl.pallas_call(
        body,
        grid=(G,),
        in_specs=[_bs((NC, BR, F2), _prow), _bs((NC, BR, CW), _prow),
                  _bs((NC, BR, CW), _prow), _bs((NC, BR, CW), _prow),
                  _bs((1, F2), _full), _bs((F3, F3), _full), _bs((1, F3), _full)],
        out_specs=_bs((1, F3), _full),
        out_shape=jax.ShapeDtypeStruct((1, F3), jnp.float32),
        scratch_shapes=[pltpu.VMEM((1, F2), jnp.float32)],
    )(agg2_p, c_p, dego_p, degi_p, b2r, W3, b3r)


def kernel(x, edge_index, W1, b1, W2, b2, W3, b3):
    src3 = edge_index[0].reshape(NW, NCHUNK, C)
    dst3 = edge_index[1].reshape(NW, NCHUNK, C)
    ones16 = jnp.ones((C, CW), jnp.float32)
    z16 = jnp.zeros((RZ, CW), jnp.float32)
    z64 = jnp.zeros((RZ, F2), jnp.float32)

    dego_p, degi_p = _sc_degrees(src3, dst3, ones16, z16)
    t1a, t1b, tnd = _tc_prep(x, W1[:, :F2], W1[:, F2:], dego_p, degi_p)
    agg1a_p = _sc_layer1_half(t1a, src3, dst3, z64)
    agg1b_p = _sc_layer1_half(t1b, src3, dst3, z64)
    t2 = _tc_mid(agg1a_p, agg1b_p, dego_p, degi_p,
                 b1[:F2].reshape(1, F2), b1[F2:].reshape(1, F2),
                 W2[:F2], W2[F2:])
    agg2_p, c_p = _sc_layer2c(t2, tnd, src3, dst3, z64, z16)
    return _tc_final(agg2_p, c_p, dego_p, degi_p,
                     b2.reshape(1, F2), W3, b3.reshape(1, F3))
